# baseline (speedup 1.0000x reference)
import functools

import jax
import jax.numpy as jnp
from jax import lax
from jax.experimental import pallas as pl
from jax.experimental.pallas import tpu as pltpu
from jax.experimental.pallas import tpu_sc as plsc

B, N, VOCAB, EMB = 16, 1024, 4096, 192
EMBP = 256
_NCK = 4
_CB = B // _NCK

_NC, _NS = 2, 16
_NW = _NC * _NS
_CHUNK = 128


def _argmax_body(x_ref, tok_ref):
    xb = x_ref[0]
    m = jnp.max(xb, axis=-1, keepdims=True)
    iota = lax.broadcasted_iota(jnp.int32, xb.shape, 1)
    idx = jnp.min(jnp.where(xb == m, iota, VOCAB), axis=-1)
    tok_ref[...] = idx.astype(jnp.int32).reshape(N // 128, 128)


def _argmax_tokens_chunk(x, c):
    return pl.pallas_call(
        _argmax_body,
        grid=(_CB,),
        in_specs=[pl.BlockSpec((1, N, VOCAB), lambda b: (c * _CB + b, 0, 0))],
        out_specs=pl.BlockSpec((N // 128, 128), lambda b: (b, 0)),
        out_shape=jax.ShapeDtypeStruct((_CB * N // 128, 128), jnp.int32),
    )(x)


_BPW = _CB * N // _NW
_WPB = N // _BPW
_NGATH = _BPW // _CHUNK
_NSLOT = min(3, _NGATH)


def _make_sc_gather():
    mesh = plsc.VectorSubcoreMesh(core_axis_name="c", subcore_axis_name="s")

    @functools.partial(
        pl.kernel,
        mesh=mesh,
        out_type=jax.ShapeDtypeStruct((_CB, N, EMBP), jnp.float32),
        scratch_types=[
            pltpu.VMEM((_NGATH, _CHUNK), jnp.int32),
            pltpu.VMEM((_NSLOT, _CHUNK, EMBP), jnp.float32),
            pltpu.SemaphoreType.DMA,
        ],
        compiler_params=pltpu.CompilerParams(use_tc_tiling_on_sc=True),
    )
    def sc_gather(table_hbm, idx_hbm, out_hbm, idx_v, rows_v, sem):
        wid = lax.axis_index("s") * _NC + lax.axis_index("c")
        b = wid // _WPB
        noff = (wid % _WPB) * _BPW
        pltpu.sync_copy(idx_hbm.at[pl.ds(wid * _NGATH, _NGATH)], idx_v)
        queue = []
        for j in range(_NGATH):
            s = j % _NSLOT
            if len(queue) == _NSLOT:
                c0, j0, s0 = queue.pop(0)
                c0.wait()
                pltpu.sync_copy(
                    rows_v.at[s0],
                    out_hbm.at[b, pl.ds(noff + j0 * _CHUNK, _CHUNK)],
                )
            queue.append(
                (
                    pltpu.async_copy(
                        table_hbm.at[idx_v.at[j]], rows_v.at[s], sem
                    ),
                    j,
                    s,
                )
            )
        for c0, j0, s0 in queue:
            c0.wait()
            pltpu.sync_copy(
                rows_v.at[s0],
                out_hbm.at[b, pl.ds(noff + j0 * _CHUNK, _CHUNK)],
            )

    return sc_gather


_SC_GATHER_CACHE = []


def kernel(x, dictionary):
    if not _SC_GATHER_CACHE:
        _SC_GATHER_CACHE.append(_make_sc_gather())
    gather = _SC_GATHER_CACHE[0]
    dict_p = jnp.pad(dictionary, ((0, 0), (0, EMBP - EMB)))
    outs = []
    for c in range(_NCK):
        tokens_c = _argmax_tokens_chunk(x, c)
        out_c = gather(dict_p, tokens_c)
        outs.append(out_c[:, :, :EMB])
    return jnp.concatenate(outs, axis=0)


# speedup vs baseline: 1.0481x; 1.0481x over previous
import functools

import jax
import jax.numpy as jnp
from jax import lax
from jax.experimental import pallas as pl
from jax.experimental.pallas import tpu as pltpu
from jax.experimental.pallas import tpu_sc as plsc

B, N, VOCAB, EMB = 16, 1024, 4096, 192
EMBP = 256
_NCK = 2
_CB = B // _NCK

_NC, _NS = 2, 16
_NW = _NC * _NS
_CHUNK = 128


def _argmax_body(x_ref, tok_ref):
    xb = x_ref[0]
    m = jnp.max(xb, axis=-1, keepdims=True)
    iota = lax.broadcasted_iota(jnp.int32, xb.shape, 1)
    idx = jnp.min(jnp.where(xb == m, iota, VOCAB), axis=-1)
    tok_ref[...] = idx.astype(jnp.int32).reshape(N // 128, 128)


def _argmax_tokens(x):
    return pl.pallas_call(
        _argmax_body,
        grid=(B,),
        in_specs=[pl.BlockSpec((1, N, VOCAB), lambda b: (b, 0, 0))],
        out_specs=pl.BlockSpec((N // 128, 128), lambda b: (b, 0)),
        out_shape=jax.ShapeDtypeStruct((B * N // 128, 128), jnp.int32),
    )(x)


_BPW = _CB * N // _NW
_WPB = N // _BPW
_NGATH = _BPW // _CHUNK
_NSLOT = min(3, _NGATH)


def _make_sc_gather(chunk):
    mesh = plsc.VectorSubcoreMesh(core_axis_name="c", subcore_axis_name="s")
    row0 = chunk * _CB * N // 128

    @functools.partial(
        pl.kernel,
        mesh=mesh,
        out_type=jax.ShapeDtypeStruct((_CB, N, EMBP), jnp.float32),
        scratch_types=[
            pltpu.VMEM((_NGATH, _CHUNK), jnp.int32),
            pltpu.VMEM((_NSLOT, _CHUNK, EMBP), jnp.float32),
            pltpu.SemaphoreType.DMA,
        ],
        compiler_params=pltpu.CompilerParams(use_tc_tiling_on_sc=True),
    )
    def sc_gather(table_hbm, idx_hbm, out_hbm, idx_v, rows_v, sem):
        wid = lax.axis_index("s") * _NC + lax.axis_index("c")
        b = wid // _WPB
        noff = (wid % _WPB) * _BPW
        pltpu.sync_copy(idx_hbm.at[pl.ds(row0 + wid * _NGATH, _NGATH)], idx_v)
        queue = []
        for j in range(_NGATH):
            s = j % _NSLOT
            if len(queue) == _NSLOT:
                c0, j0, s0 = queue.pop(0)
                c0.wait()
                pltpu.sync_copy(
                    rows_v.at[s0],
                    out_hbm.at[b, pl.ds(noff + j0 * _CHUNK, _CHUNK)],
                )
            queue.append(
                (
                    pltpu.async_copy(
                        table_hbm.at[idx_v.at[j]], rows_v.at[s], sem
                    ),
                    j,
                    s,
                )
            )
        for c0, j0, s0 in queue:
            c0.wait()
            pltpu.sync_copy(
                rows_v.at[s0],
                out_hbm.at[b, pl.ds(noff + j0 * _CHUNK, _CHUNK)],
            )

    return sc_gather


def _narrow_body(g_ref, o_ref):
    o_ref[...] = g_ref[:, :, :EMB]


def _narrow_first(g):
    return pl.pallas_call(
        _narrow_body,
        grid=(_CB,),
        in_specs=[pl.BlockSpec((1, N, EMBP), lambda b: (b, 0, 0))],
        out_specs=pl.BlockSpec((1, N, EMB), lambda b: (b, 0, 0)),
        out_shape=jax.ShapeDtypeStruct((B, N, EMB), jnp.float32),
    )(g)


def _narrow_next_body(g_ref, _, o_ref):
    o_ref[...] = g_ref[:, :, :EMB]


def _narrow_next(g, acc, chunk):
    return pl.pallas_call(
        _narrow_next_body,
        grid=(_CB,),
        in_specs=[
            pl.BlockSpec((1, N, EMBP), lambda b: (b, 0, 0)),
            pl.BlockSpec(memory_space=pl.ANY),
        ],
        out_specs=pl.BlockSpec((1, N, EMB), lambda b: (chunk * _CB + b, 0, 0)),
        out_shape=jax.ShapeDtypeStruct((B, N, EMB), jnp.float32),
        input_output_aliases={1: 0},
    )(g, acc)


_SC_GATHER_CACHE = {}


def kernel(x, dictionary):
    if not _SC_GATHER_CACHE:
        for c in range(_NCK):
            _SC_GATHER_CACHE[c] = _make_sc_gather(c)
    tokens = _argmax_tokens(x)
    dict_p = jnp.pad(dictionary, ((0, 0), (0, EMBP - EMB)))
    slabs = [_SC_GATHER_CACHE[c](dict_p, tokens) for c in range(_NCK)]
    out = _narrow_first(slabs[0])
    for c in range(1, _NCK):
        out = _narrow_next(slabs[c], out, c)
    return out


# speedup vs baseline: 1.1832x; 1.1289x over previous
import functools

import jax
import jax.numpy as jnp
from jax import lax
from jax.experimental import pallas as pl
from jax.experimental.pallas import tpu as pltpu
from jax.experimental.pallas import tpu_sc as plsc

B, N, VOCAB, EMB = 16, 1024, 4096, 192
EMBP = 256

_NC, _NS = 2, 16
_NW = _NC * _NS
_CHUNK = 128


def _argmax_body(x_ref, tok_ref):
    xb = x_ref[0]
    m = jnp.max(xb, axis=-1, keepdims=True)
    iota = lax.broadcasted_iota(jnp.int32, xb.shape, 1)
    idx = jnp.min(jnp.where(xb == m, iota, VOCAB), axis=-1)
    tok_ref[...] = idx.astype(jnp.int32).reshape(N // 128, 128)


def _argmax_tokens(x):
    return pl.pallas_call(
        _argmax_body,
        grid=(B,),
        in_specs=[pl.BlockSpec((1, N, VOCAB), lambda b: (b, 0, 0))],
        out_specs=pl.BlockSpec((N // 128, 128), lambda b: (b, 0)),
        out_shape=jax.ShapeDtypeStruct((B * N // 128, 128), jnp.int32),
    )(x)


_BPW = B * N // _NW
_WPB = N // _BPW
_NGATH = _BPW // _CHUNK
_NSLOT = 3


def _make_sc_gather():
    mesh = plsc.VectorSubcoreMesh(core_axis_name="c", subcore_axis_name="s")

    @functools.partial(
        pl.kernel,
        mesh=mesh,
        out_type=jax.ShapeDtypeStruct((B, N, EMBP), jnp.float32),
        scratch_types=[
            pltpu.VMEM((_NGATH, _CHUNK), jnp.int32),
            pltpu.VMEM((_NSLOT, _CHUNK, EMBP), jnp.float32),
            pltpu.SemaphoreType.DMA,
            pltpu.SemaphoreType.DMA,
        ],
        compiler_params=pltpu.CompilerParams(use_tc_tiling_on_sc=True),
    )
    def sc_gather(table_hbm, idx_hbm, out_hbm, idx_v, rows_v, gsem, wsem):
        wid = lax.axis_index("s") * _NC + lax.axis_index("c")
        b = wid // _WPB
        noff = (wid % _WPB) * _BPW
        pltpu.sync_copy(idx_hbm.at[pl.ds(wid * _NGATH, _NGATH)], idx_v)
        gathers = []
        writes = []
        for j in range(_NGATH):
            s = j % _NSLOT
            if j >= _NSLOT:
                writes[j - _NSLOT].wait()
            gathers.append(
                pltpu.async_copy(table_hbm.at[idx_v.at[j]], rows_v.at[s], gsem)
            )
            jd = j - _NSLOT + 1
            if jd >= 0:
                gathers[jd].wait()
                writes.append(
                    pltpu.async_copy(
                        rows_v.at[jd % _NSLOT],
                        out_hbm.at[b, pl.ds(noff + jd * _CHUNK, _CHUNK)],
                        wsem,
                    )
                )
        for jd in range(_NGATH - _NSLOT + 1, _NGATH):
            gathers[jd].wait()
            writes.append(
                pltpu.async_copy(
                    rows_v.at[jd % _NSLOT],
                    out_hbm.at[b, pl.ds(noff + jd * _CHUNK, _CHUNK)],
                    wsem,
                )
            )
        for w in writes[max(0, _NGATH - _NSLOT):]:
            w.wait()

    return sc_gather


_SC_GATHER_CACHE = []


def kernel(x, dictionary):
    if not _SC_GATHER_CACHE:
        _SC_GATHER_CACHE.append(_make_sc_gather())
    tokens = _argmax_tokens(x)
    dict_p = jnp.pad(dictionary, ((0, 0), (0, EMBP - EMB)))
    out_p = _SC_GATHER_CACHE[0](dict_p, tokens)
    return out_p[:, :, :EMB]
